# baseline (speedup 1.0000x reference)
import functools

import jax
import jax.numpy as jnp
from jax import lax
from jax.experimental import pallas as pl
from jax.experimental.pallas import tpu as pltpu
from jax.experimental.pallas import tpu_sc as plsc

N_NODES = 10000
N_EDGES = 320000
IN_DIM = 128
OUT_DIM = 128

NC = 2
NS = 16
CHUNK = 128
NPHASE = 2
NGROUP = 40
EDGES_PER_TILE = NPHASE * NGROUP * CHUNK
PAD_EDGES = NC * NS * EDGES_PER_TILE
ROWS_PER_TILE = 640
ACC_ROWS = NS * ROWS_PER_TILE


def _mm_body(x_ref, w_ref, o_ref):
    o_ref[...] = jnp.dot(x_ref[...], w_ref[...], preferred_element_type=jnp.float32)


def _matmul(x, w):
    grid = 10
    blk = N_NODES // grid
    return pl.pallas_call(
        _mm_body,
        grid=(grid,),
        in_specs=[
            pl.BlockSpec((blk, IN_DIM), lambda i: (i, 0)),
            pl.BlockSpec((IN_DIM, OUT_DIM), lambda i: (0, 0)),
        ],
        out_specs=pl.BlockSpec((blk, OUT_DIM), lambda i: (i, 0)),
        out_shape=jax.ShapeDtypeStruct((N_NODES, OUT_DIM), jnp.float32),
    )(x, w)


_sc_mesh = plsc.VectorSubcoreMesh(
    core_axis_name="c", subcore_axis_name="s", num_cores=NC, num_subcores=NS
)


@functools.partial(
    pl.kernel,
    out_type=jax.ShapeDtypeStruct((NC * ACC_ROWS, OUT_DIM), jnp.float32),
    mesh=_sc_mesh,
    scratch_types=[
        pltpu.VMEM((NGROUP, 2, CHUNK), jnp.int32),
        pltpu.VMEM((CHUNK, OUT_DIM), jnp.float32),
        pltpu.VMEM_SHARED((ACC_ROWS, OUT_DIM), jnp.float32),
        pltpu.SemaphoreType.DMA,
    ],
)
def _sc_aggregate(epk_hbm, h_hbm, z_hbm, out_hbm, idx_v, rows_a, acc, sem_a):
    c = lax.axis_index("c")
    s = lax.axis_index("s")
    wid = c * NS + s

    pltpu.sync_copy(z_hbm, rows_a.at[pl.ds(0, CHUNK)])
    for k in range(ROWS_PER_TILE // CHUNK):
        pltpu.sync_copy(
            rows_a.at[pl.ds(0, CHUNK)],
            acc.at[pl.ds(s * ROWS_PER_TILE + k * CHUNK, CHUNK)],
        )

    def body(g, carry):
        pltpu.async_copy(h_hbm.at[idx_v.at[g, 0]], rows_a, sem_a).wait()
        pltpu.sync_copy(rows_a, acc.at[idx_v.at[g, 1]], add=True)
        return carry

    plsc.subcore_barrier()
    for ph in range(NPHASE):
        pltpu.sync_copy(epk_hbm.at[wid, ph], idx_v)
        lax.fori_loop(0, NGROUP, body, 0)
    plsc.subcore_barrier()

    pltpu.sync_copy(
        acc.at[pl.ds(s * ROWS_PER_TILE, ROWS_PER_TILE)],
        out_hbm.at[pl.ds(c * ACC_ROWS + s * ROWS_PER_TILE, ROWS_PER_TILE)],
    )


def _combine_body(p_ref, o_ref):
    o_ref[...] = jnp.maximum(p_ref[0] + p_ref[1], 0.0)


def _combine(partials):
    grid = 10
    blk = N_NODES // grid
    return pl.pallas_call(
        _combine_body,
        grid=(grid,),
        in_specs=[pl.BlockSpec((NC, blk, OUT_DIM), lambda i: (0, i, 0))],
        out_specs=pl.BlockSpec((blk, OUT_DIM), lambda i: (i, 0)),
        out_shape=jax.ShapeDtypeStruct((N_NODES, OUT_DIM), jnp.float32),
    )(partials)


def kernel(x, edge_index, W):
    ei = edge_index.astype(jnp.int32)
    dst = ei[0]
    src = ei[1]
    pad = PAD_EDGES - N_EDGES
    src_p = jnp.concatenate([src, jnp.zeros((pad,), jnp.int32)])
    dump_rows = N_NODES + jnp.arange(pad, dtype=jnp.int32) % (ACC_ROWS - N_NODES)
    dst_p = jnp.concatenate([dst, dump_rows])
    shp = (NC * NS, NPHASE, NGROUP, CHUNK)
    epk = jnp.stack([src_p.reshape(shp), dst_p.reshape(shp)], axis=3)
    zeros_rows = jnp.zeros((CHUNK, OUT_DIM), jnp.float32)

    h = _matmul(x, W)
    partials = _sc_aggregate(epk, h, zeros_rows)
    p2 = partials.reshape(NC, ACC_ROWS, OUT_DIM)[:, :N_NODES, :]
    return _combine(p2)


# speedup vs baseline: 1.5096x; 1.5096x over previous
import functools

import jax
import jax.numpy as jnp
from jax import lax
from jax.experimental import pallas as pl
from jax.experimental.pallas import tpu as pltpu
from jax.experimental.pallas import tpu_sc as plsc

N_NODES = 10000
N_EDGES = 320000
IN_DIM = 128
OUT_DIM = 128

NC = 2
NS = 16
CHUNK = 128
NPHASE = 1
NGROUP = 79
EDGES_PER_TILE = NPHASE * NGROUP * CHUNK
PAD_EDGES = NC * NS * EDGES_PER_TILE
ROWS_PER_TILE = 640
ACC_ROWS = NS * ROWS_PER_TILE


def _mm_body(x_ref, w_ref, o_ref):
    o_ref[...] = jnp.dot(x_ref[...], w_ref[...], preferred_element_type=jnp.float32)


def _matmul(x, w):
    grid = 10
    blk = N_NODES // grid
    return pl.pallas_call(
        _mm_body,
        grid=(grid,),
        in_specs=[
            pl.BlockSpec((blk, IN_DIM), lambda i: (i, 0)),
            pl.BlockSpec((IN_DIM, OUT_DIM), lambda i: (0, 0)),
        ],
        out_specs=pl.BlockSpec((blk, OUT_DIM), lambda i: (i, 0)),
        out_shape=jax.ShapeDtypeStruct((N_NODES, OUT_DIM), jnp.float32),
    )(x, w)


_sc_mesh = plsc.VectorSubcoreMesh(
    core_axis_name="c", subcore_axis_name="s", num_cores=NC, num_subcores=NS
)


@functools.partial(
    pl.kernel,
    out_type=jax.ShapeDtypeStruct((NC * ACC_ROWS, OUT_DIM), jnp.float32),
    mesh=_sc_mesh,
    scratch_types=[
        pltpu.VMEM((NGROUP, 2, CHUNK), jnp.int32),
        pltpu.VMEM((CHUNK, OUT_DIM), jnp.float32),
        pltpu.VMEM_SHARED((ACC_ROWS, OUT_DIM), jnp.float32),
        pltpu.SemaphoreType.DMA,
    ],
)
def _sc_aggregate(epk_hbm, h_hbm, z_hbm, out_hbm, idx_v, rows_a, acc, sem_a):
    c = lax.axis_index("c")
    s = lax.axis_index("s")
    wid = c * NS + s

    pltpu.sync_copy(z_hbm, rows_a.at[pl.ds(0, CHUNK)])
    for k in range(ROWS_PER_TILE // CHUNK):
        pltpu.sync_copy(
            rows_a.at[pl.ds(0, CHUNK)],
            acc.at[pl.ds(s * ROWS_PER_TILE + k * CHUNK, CHUNK)],
        )

    def body(g, carry):
        pltpu.async_copy(h_hbm.at[idx_v.at[g, 0]], rows_a, sem_a).wait()
        pltpu.sync_copy(rows_a, acc.at[idx_v.at[g, 1]], add=True)
        return carry

    plsc.subcore_barrier()
    for ph in range(NPHASE):
        pltpu.sync_copy(epk_hbm.at[wid, ph], idx_v)
        lax.fori_loop(0, NGROUP, body, 0)
    plsc.subcore_barrier()

    pltpu.sync_copy(
        acc.at[pl.ds(s * ROWS_PER_TILE, ROWS_PER_TILE)],
        out_hbm.at[pl.ds(c * ACC_ROWS + s * ROWS_PER_TILE, ROWS_PER_TILE)],
    )


def _combine_body(p_ref, o_ref):
    o_ref[...] = jnp.maximum(p_ref[0] + p_ref[1], 0.0)


def _combine(partials):
    grid = 10
    blk = N_NODES // grid
    return pl.pallas_call(
        _combine_body,
        grid=(grid,),
        in_specs=[pl.BlockSpec((NC, blk, OUT_DIM), lambda i: (0, i, 0))],
        out_specs=pl.BlockSpec((blk, OUT_DIM), lambda i: (i, 0)),
        out_shape=jax.ShapeDtypeStruct((N_NODES, OUT_DIM), jnp.float32),
    )(partials)


def kernel(x, edge_index, W):
    ei = edge_index.astype(jnp.int32)
    dst = ei[0]
    src = ei[1]
    pad = PAD_EDGES - N_EDGES
    src_p = jnp.concatenate([src, jnp.zeros((pad,), jnp.int32)])
    dump_rows = N_NODES + jnp.arange(pad, dtype=jnp.int32) % (ACC_ROWS - N_NODES)
    dst_p = jnp.concatenate([dst, dump_rows])
    shp = (NC * NS, NPHASE, NGROUP, CHUNK)
    epk = jnp.stack([src_p.reshape(shp), dst_p.reshape(shp)], axis=3)
    zeros_rows = jnp.zeros((CHUNK, OUT_DIM), jnp.float32)

    h = _matmul(x, W)
    partials = _sc_aggregate(epk, h, zeros_rows)
    p2 = partials.reshape(NC, ACC_ROWS, OUT_DIM)[:, :N_NODES, :]
    return _combine(p2)


# speedup vs baseline: 1.7716x; 1.1736x over previous
import functools

import jax
import jax.numpy as jnp
from jax import lax
from jax.experimental import pallas as pl
from jax.experimental.pallas import tpu as pltpu
from jax.experimental.pallas import tpu_sc as plsc

N_NODES = 10000
N_EDGES = 320000
IN_DIM = 128
OUT_DIM = 128

NC = 2
NS = 16
CHUNK = 128
N0 = 56
N1 = 101
PAD_EDGES = NS * (N0 + N1) * CHUNK
ROWS_PER_TILE = 640
ACC_ROWS = NS * ROWS_PER_TILE


def _mm_body(x_ref, w_ref, o_ref):
    o_ref[...] = jnp.dot(x_ref[...], w_ref[...], preferred_element_type=jnp.float32)


def _matmul(x, w):
    grid = 10
    blk = N_NODES // grid
    return pl.pallas_call(
        _mm_body,
        grid=(grid,),
        in_specs=[
            pl.BlockSpec((blk, IN_DIM), lambda i: (i, 0)),
            pl.BlockSpec((IN_DIM, OUT_DIM), lambda i: (0, 0)),
        ],
        out_specs=pl.BlockSpec((blk, OUT_DIM), lambda i: (i, 0)),
        out_shape=jax.ShapeDtypeStruct((N_NODES, OUT_DIM), jnp.float32),
    )(x, w)


_sc_mesh = plsc.VectorSubcoreMesh(
    core_axis_name="c", subcore_axis_name="s", num_cores=NC, num_subcores=NS
)


@functools.partial(
    pl.kernel,
    out_type=jax.ShapeDtypeStruct((NC * ACC_ROWS, OUT_DIM), jnp.float32),
    mesh=_sc_mesh,
    scratch_types=[
        pltpu.VMEM((N1, 2, CHUNK), jnp.int32),
        pltpu.VMEM((CHUNK, OUT_DIM), jnp.float32),
        pltpu.VMEM_SHARED((ACC_ROWS, OUT_DIM), jnp.float32),
        pltpu.SemaphoreType.DMA,
    ],
)
def _sc_aggregate(epk_hbm, h_hbm, z_hbm, out_hbm, idx_v, rows_a, acc, sem_a):
    c = lax.axis_index("c")
    s = lax.axis_index("s")
    wid = c * NS + s

    pltpu.sync_copy(z_hbm, rows_a.at[pl.ds(0, CHUNK)])
    for k in range(ROWS_PER_TILE // CHUNK):
        pltpu.sync_copy(
            rows_a.at[pl.ds(0, CHUNK)],
            acc.at[pl.ds(s * ROWS_PER_TILE + k * CHUNK, CHUNK)],
        )

    def body(g, carry):
        pltpu.async_copy(h_hbm.at[idx_v.at[g, 0]], rows_a, sem_a).wait()
        pltpu.sync_copy(rows_a, acc.at[idx_v.at[g, 1]], add=True)
        return carry

    pltpu.sync_copy(epk_hbm.at[wid], idx_v)
    plsc.subcore_barrier()

    @pl.when(c == 0)
    def _():
        lax.fori_loop(0, N0, body, 0)

    @pl.when(c == 1)
    def _():
        lax.fori_loop(0, N1, body, 0)

    plsc.subcore_barrier()

    pltpu.sync_copy(
        acc.at[pl.ds(s * ROWS_PER_TILE, ROWS_PER_TILE)],
        out_hbm.at[pl.ds(c * ACC_ROWS + s * ROWS_PER_TILE, ROWS_PER_TILE)],
    )


def _combine_body(p_ref, o_ref):
    o_ref[...] = jnp.maximum(p_ref[0] + p_ref[1], 0.0)


def _combine(partials):
    grid = 10
    blk = N_NODES // grid
    return pl.pallas_call(
        _combine_body,
        grid=(grid,),
        in_specs=[pl.BlockSpec((NC, blk, OUT_DIM), lambda i: (0, i, 0))],
        out_specs=pl.BlockSpec((blk, OUT_DIM), lambda i: (i, 0)),
        out_shape=jax.ShapeDtypeStruct((N_NODES, OUT_DIM), jnp.float32),
    )(partials)


def kernel(x, edge_index, W):
    ei = edge_index.astype(jnp.int32)
    dst = ei[0]
    src = ei[1]
    pad = PAD_EDGES - N_EDGES
    src_p = jnp.concatenate([src, jnp.zeros((pad,), jnp.int32)])
    dump_rows = N_NODES + jnp.arange(pad, dtype=jnp.int32) % (ACC_ROWS - N_NODES)
    dst_p = jnp.concatenate([dst, dump_rows])
    n0e = NS * N0 * CHUNK
    def pack(a):
        a0 = a[:n0e].reshape(NS, N0, CHUNK)
        a0 = jnp.pad(a0, ((0, 0), (0, N1 - N0), (0, 0)))
        a1 = a[n0e:].reshape(NS, N1, CHUNK)
        return jnp.concatenate([a0, a1], axis=0)
    epk = jnp.stack([pack(src_p), pack(dst_p)], axis=2)
    zeros_rows = jnp.zeros((CHUNK, OUT_DIM), jnp.float32)

    h = _matmul(x, W)
    partials = _sc_aggregate(epk, h, zeros_rows)
    p2 = partials.reshape(NC, ACC_ROWS, OUT_DIM)[:, :N_NODES, :]
    return _combine(p2)


# speedup vs baseline: 1.8932x; 1.0686x over previous
import functools

import jax
import jax.numpy as jnp
from jax import lax
from jax.experimental import pallas as pl
from jax.experimental.pallas import tpu as pltpu
from jax.experimental.pallas import tpu_sc as plsc

N_NODES = 10000
N_EDGES = 320000
IN_DIM = 128
OUT_DIM = 128

NC = 2
NS = 16
CHUNK = 128
N0 = 83
N1 = 74
PAD_EDGES = NS * (N0 + N1) * CHUNK
ROWS_PER_TILE = 640
ACC_ROWS = NS * ROWS_PER_TILE


def _mm_body(x_ref, w_ref, o_ref):
    o_ref[...] = jnp.dot(x_ref[...], w_ref[...], preferred_element_type=jnp.float32)


def _matmul(x, w):
    grid = 10
    blk = N_NODES // grid
    return pl.pallas_call(
        _mm_body,
        grid=(grid,),
        in_specs=[
            pl.BlockSpec((blk, IN_DIM), lambda i: (i, 0)),
            pl.BlockSpec((IN_DIM, OUT_DIM), lambda i: (0, 0)),
        ],
        out_specs=pl.BlockSpec((blk, OUT_DIM), lambda i: (i, 0)),
        out_shape=jax.ShapeDtypeStruct((N_NODES, OUT_DIM), jnp.float32),
    )(x, w)


_sc_mesh = plsc.VectorSubcoreMesh(
    core_axis_name="c", subcore_axis_name="s", num_cores=NC, num_subcores=NS
)


@functools.partial(
    pl.kernel,
    out_type=jax.ShapeDtypeStruct((NC * ACC_ROWS, OUT_DIM), jnp.float32),
    mesh=_sc_mesh,
    scratch_types=[
        pltpu.VMEM((max(N0, N1), 2, CHUNK), jnp.int32),
        pltpu.VMEM((CHUNK, OUT_DIM), jnp.float32),
        pltpu.VMEM_SHARED((ACC_ROWS, OUT_DIM), jnp.float32),
        pltpu.SemaphoreType.DMA,
    ],
)
def _sc_aggregate(epk_hbm, h_hbm, z_hbm, out_hbm, idx_v, rows_a, acc, sem_a):
    c = lax.axis_index("c")
    s = lax.axis_index("s")
    wid = c * NS + s

    pltpu.sync_copy(z_hbm, rows_a.at[pl.ds(0, CHUNK)])
    for k in range(ROWS_PER_TILE // CHUNK):
        pltpu.sync_copy(
            rows_a.at[pl.ds(0, CHUNK)],
            acc.at[pl.ds(s * ROWS_PER_TILE + k * CHUNK, CHUNK)],
        )

    def body(g, carry):
        pltpu.async_copy(h_hbm.at[idx_v.at[g, 0]], rows_a, sem_a).wait()
        pltpu.sync_copy(rows_a, acc.at[idx_v.at[g, 1]], add=True)
        return carry

    pltpu.sync_copy(epk_hbm.at[wid], idx_v)
    plsc.subcore_barrier()

    @pl.when(c == 0)
    def _():
        lax.fori_loop(0, N0, body, 0)

    @pl.when(c == 1)
    def _():
        lax.fori_loop(0, N1, body, 0)

    plsc.subcore_barrier()

    pltpu.sync_copy(
        acc.at[pl.ds(s * ROWS_PER_TILE, ROWS_PER_TILE)],
        out_hbm.at[pl.ds(c * ACC_ROWS + s * ROWS_PER_TILE, ROWS_PER_TILE)],
    )


def _combine_body(p_ref, o_ref):
    o_ref[...] = jnp.maximum(p_ref[0] + p_ref[1], 0.0)


def _combine(partials):
    grid = 10
    blk = N_NODES // grid
    return pl.pallas_call(
        _combine_body,
        grid=(grid,),
        in_specs=[pl.BlockSpec((NC, blk, OUT_DIM), lambda i: (0, i, 0))],
        out_specs=pl.BlockSpec((blk, OUT_DIM), lambda i: (i, 0)),
        out_shape=jax.ShapeDtypeStruct((N_NODES, OUT_DIM), jnp.float32),
    )(partials)


def kernel(x, edge_index, W):
    ei = edge_index.astype(jnp.int32)
    dst = ei[0]
    src = ei[1]
    pad = PAD_EDGES - N_EDGES
    src_p = jnp.concatenate([src, jnp.zeros((pad,), jnp.int32)])
    dump_rows = N_NODES + jnp.arange(pad, dtype=jnp.int32) % (ACC_ROWS - N_NODES)
    dst_p = jnp.concatenate([dst, dump_rows])
    n0e = NS * N0 * CHUNK
    def pack(a):
        m = max(N0, N1)
        a0 = a[:n0e].reshape(NS, N0, CHUNK)
        a0 = jnp.pad(a0, ((0, 0), (0, m - N0), (0, 0)))
        a1 = a[n0e:].reshape(NS, N1, CHUNK)
        a1 = jnp.pad(a1, ((0, 0), (0, m - N1), (0, 0)))
        return jnp.concatenate([a0, a1], axis=0)
    epk = jnp.stack([pack(src_p), pack(dst_p)], axis=2)
    zeros_rows = jnp.zeros((CHUNK, OUT_DIM), jnp.float32)

    h = _matmul(x, W)
    partials = _sc_aggregate(epk, h, zeros_rows)
    p2 = partials.reshape(NC, ACC_ROWS, OUT_DIM)[:, :N_NODES, :]
    return _combine(p2)


# speedup vs baseline: 1.9963x; 1.0545x over previous
import functools

import jax
import jax.numpy as jnp
from jax import lax
from jax.experimental import pallas as pl
from jax.experimental.pallas import tpu as pltpu
from jax.experimental.pallas import tpu_sc as plsc

N_NODES = 10000
N_EDGES = 320000
IN_DIM = 128
OUT_DIM = 128

NC = 2
NS = 16
CHUNK = 128
N0 = 92
N1 = 65
PAD_EDGES = NS * (N0 + N1) * CHUNK
ROWS_PER_TILE = 640
ACC_ROWS = NS * ROWS_PER_TILE


def _mm_body(x_ref, w_ref, o_ref):
    o_ref[...] = jnp.dot(x_ref[...], w_ref[...], preferred_element_type=jnp.float32)


def _matmul(x, w):
    grid = 10
    blk = N_NODES // grid
    return pl.pallas_call(
        _mm_body,
        grid=(grid,),
        in_specs=[
            pl.BlockSpec((blk, IN_DIM), lambda i: (i, 0)),
            pl.BlockSpec((IN_DIM, OUT_DIM), lambda i: (0, 0)),
        ],
        out_specs=pl.BlockSpec((blk, OUT_DIM), lambda i: (i, 0)),
        out_shape=jax.ShapeDtypeStruct((N_NODES, OUT_DIM), jnp.float32),
    )(x, w)


_sc_mesh = plsc.VectorSubcoreMesh(
    core_axis_name="c", subcore_axis_name="s", num_cores=NC, num_subcores=NS
)


@functools.partial(
    pl.kernel,
    out_type=jax.ShapeDtypeStruct((NC * ACC_ROWS, OUT_DIM), jnp.float32),
    mesh=_sc_mesh,
    scratch_types=[
        pltpu.VMEM((max(N0, N1), 2, CHUNK), jnp.int32),
        pltpu.VMEM((CHUNK, OUT_DIM), jnp.float32),
        pltpu.VMEM_SHARED((ACC_ROWS, OUT_DIM), jnp.float32),
        pltpu.SemaphoreType.DMA,
    ],
)
def _sc_aggregate(epk_hbm, h_hbm, z_hbm, out_hbm, idx_v, rows_a, acc, sem_a):
    c = lax.axis_index("c")
    s = lax.axis_index("s")
    wid = c * NS + s

    pltpu.sync_copy(z_hbm, rows_a.at[pl.ds(0, CHUNK)])
    for k in range(ROWS_PER_TILE // CHUNK):
        pltpu.sync_copy(
            rows_a.at[pl.ds(0, CHUNK)],
            acc.at[pl.ds(s * ROWS_PER_TILE + k * CHUNK, CHUNK)],
        )

    def body(g, carry):
        pltpu.async_copy(h_hbm.at[idx_v.at[g, 0]], rows_a, sem_a).wait()
        pltpu.sync_copy(rows_a, acc.at[idx_v.at[g, 1]], add=True)
        return carry

    pltpu.sync_copy(epk_hbm.at[wid], idx_v)
    plsc.subcore_barrier()

    @pl.when(c == 0)
    def _():
        lax.fori_loop(0, N0, body, 0)

    @pl.when(c == 1)
    def _():
        lax.fori_loop(0, N1, body, 0)

    plsc.subcore_barrier()

    pltpu.sync_copy(
        acc.at[pl.ds(s * ROWS_PER_TILE, ROWS_PER_TILE)],
        out_hbm.at[pl.ds(c * ACC_ROWS + s * ROWS_PER_TILE, ROWS_PER_TILE)],
    )


def _combine_body(p_ref, o_ref):
    o_ref[...] = jnp.maximum(p_ref[0] + p_ref[1], 0.0)


def _combine(partials):
    grid = 10
    blk = N_NODES // grid
    return pl.pallas_call(
        _combine_body,
        grid=(grid,),
        in_specs=[pl.BlockSpec((NC, blk, OUT_DIM), lambda i: (0, i, 0))],
        out_specs=pl.BlockSpec((blk, OUT_DIM), lambda i: (i, 0)),
        out_shape=jax.ShapeDtypeStruct((N_NODES, OUT_DIM), jnp.float32),
    )(partials)


def kernel(x, edge_index, W):
    ei = edge_index.astype(jnp.int32)
    dst = ei[0]
    src = ei[1]
    pad = PAD_EDGES - N_EDGES
    src_p = jnp.concatenate([src, jnp.zeros((pad,), jnp.int32)])
    dump_rows = N_NODES + jnp.arange(pad, dtype=jnp.int32) % (ACC_ROWS - N_NODES)
    dst_p = jnp.concatenate([dst, dump_rows])
    n0e = NS * N0 * CHUNK
    def pack(a):
        m = max(N0, N1)
        a0 = a[:n0e].reshape(NS, N0, CHUNK)
        a0 = jnp.pad(a0, ((0, 0), (0, m - N0), (0, 0)))
        a1 = a[n0e:].reshape(NS, N1, CHUNK)
        a1 = jnp.pad(a1, ((0, 0), (0, m - N1), (0, 0)))
        return jnp.concatenate([a0, a1], axis=0)
    epk = jnp.stack([pack(src_p), pack(dst_p)], axis=2)
    zeros_rows = jnp.zeros((CHUNK, OUT_DIM), jnp.float32)

    h = _matmul(x, W)
    partials = _sc_aggregate(epk, h, zeros_rows)
    p2 = partials.reshape(NC, ACC_ROWS, OUT_DIM)[:, :N_NODES, :]
    return _combine(p2)
